# Initial kernel scaffold; baseline (speedup 1.0000x reference)
import jax
import jax.numpy as jnp
from jax.experimental import pallas as pl


def kernel(x, edge_index, edge_w, W0, W1, bias):
    raise NotImplementedError("write your pallas kernel here")


# speedup vs baseline: 260.3326x; 260.3326x over previous
import functools

import jax
import jax.numpy as jnp
from jax import lax
from jax.experimental import pallas as pl
from jax.experimental.pallas import tpu as pltpu
from jax.experimental.pallas import tpu_sc as plsc

_N = 10000
_E = 320000
_B = 2
_F = 128
_NT = 16
_NC = 2
_CHUNK = _E // _NT
_HALF = _CHUNK // _NC
_NPAD = 10240
_SLICE = _NPAD // _NT
_BN = 2000
_NB = _N // _BN


def _rsqrt16(d):
    i = plsc.bitcast(d, jnp.int32)
    i = jnp.int32(0x5F3759DF) - (i >> 1)
    y = plsc.bitcast(i, jnp.float32)
    for _ in range(3):
        y = y * (1.5 - 0.5 * d * y * y)
    return jnp.where(d > 0.0, y, 0.0)


def _edge_body(src_hbm, dst_hbm, w_hbm, z1_hbm, t_hbm,
               src_v, dst_v, w_v, z1_v, dinv_v, t_v, red_v, slc_v,
               deg_sh, dinv_sh, t_sh):
    c = lax.axis_index("c")
    s = lax.axis_index("s")

    pltpu.sync_copy(z1_hbm, z1_v)

    def _zero_deg(i, carry):
        t_v[pl.ds(i * 16, 16)] = jnp.zeros((16,), jnp.float32)
        return carry
    lax.fori_loop(0, _N // 16, _zero_deg, None)

    def _deg(i, carry):
        sv = src_v[pl.ds(i * 16, 16)]
        dv = dst_v[pl.ds(i * 16, 16)]
        wv = w_v[pl.ds(i * 16, 16)]
        wz = jnp.where(sv != dv, wv, 0.0)
        plsc.addupdate_scatter(t_v, [sv], wz)
        return carry

    for p in range(_CHUNK // _HALF):
        off = s * _CHUNK + p * _HALF
        pltpu.sync_copy(src_hbm.at[pl.ds(off, _HALF)], src_v)
        pltpu.sync_copy(dst_hbm.at[pl.ds(off, _HALF)], dst_v)
        pltpu.sync_copy(w_hbm.at[pl.ds(off, _HALF)], w_v)
        lax.fori_loop(0, _HALF // 16, _deg, None)

    pltpu.sync_copy(t_v.at[pl.ds(0, _N)], deg_sh.at[s, pl.ds(0, _N)])
    plsc.subcore_barrier()

    pltpu.sync_copy(deg_sh.at[:, pl.ds(s * _SLICE, _SLICE)], red_v)

    def _dinv(j, carry):
        acc = red_v[0, pl.ds(j * 16, 16)]
        for k in range(1, _NT):
            acc = acc + red_v[k, pl.ds(j * 16, 16)]
        slc_v[pl.ds(j * 16, 16)] = _rsqrt16(acc)
        return carry
    lax.fori_loop(0, _SLICE // 16, _dinv, None)

    pltpu.sync_copy(slc_v, dinv_sh.at[pl.ds(s * _SLICE, _SLICE)])
    plsc.subcore_barrier()
    pltpu.sync_copy(dinv_sh.at[pl.ds(0, _N)], dinv_v)

    def _zero_t(i, carry):
        t_v[pl.ds(i * 16, 16)] = jnp.zeros((16,), jnp.float32)
        return carry
    lax.fori_loop(0, (_B * _N) // 16, _zero_t, None)

    moff = s * _CHUNK + c * _HALF
    pltpu.sync_copy(src_hbm.at[pl.ds(moff, _HALF)], src_v)
    pltpu.sync_copy(dst_hbm.at[pl.ds(moff, _HALF)], dst_v)
    pltpu.sync_copy(w_hbm.at[pl.ds(moff, _HALF)], w_v)

    def _msg(i, carry):
        off = i * 16
        sv = src_v[pl.ds(off, 16)]
        dv = dst_v[pl.ds(off, 16)]
        wv = w_v[pl.ds(off, 16)]
        dsrc = plsc.load_gather(dinv_v, [sv])
        ddst = plsc.load_gather(dinv_v, [dv])
        ce = jnp.where(sv != dv, -(dsrc * wv * ddst), 0.0)
        g0 = plsc.load_gather(z1_v, [sv])
        g1 = plsc.load_gather(z1_v, [sv + _N])
        plsc.addupdate_scatter(t_v, [dv], ce * g0)
        plsc.addupdate_scatter(t_v, [dv + _N], ce * g1)
        return carry
    lax.fori_loop(0, _HALF // 16, _msg, None)

    pltpu.sync_copy(t_v.at[pl.ds(0, _N)], t_sh.at[s, 0, pl.ds(0, _N)])
    pltpu.sync_copy(t_v.at[pl.ds(_N, _N)], t_sh.at[s, 1, pl.ds(0, _N)])
    plsc.subcore_barrier()

    for b in range(_B):
        pltpu.sync_copy(t_sh.at[:, b, pl.ds(s * _SLICE, _SLICE)], red_v)

        def _tred(j, carry):
            acc = red_v[0, pl.ds(j * 16, 16)]
            for k in range(1, _NT):
                acc = acc + red_v[k, pl.ds(j * 16, 16)]
            slc_v[pl.ds(j * 16, 16)] = acc
            return carry
        lax.fori_loop(0, _SLICE // 16, _tred, None)
        pltpu.sync_copy(slc_v, t_hbm.at[c, b, pl.ds(s * _SLICE, _SLICE)])


_edge_sc = functools.partial(
    pl.kernel,
    out_type=jax.ShapeDtypeStruct((_NC, _B, _NPAD), jnp.float32),
    mesh=plsc.VectorSubcoreMesh(core_axis_name="c", subcore_axis_name="s"),
    compiler_params=pltpu.CompilerParams(use_tc_tiling_on_sc=False,
                                         needs_layout_passes=False),
    scratch_types=[
        pltpu.VMEM((_HALF,), jnp.int32),
        pltpu.VMEM((_HALF,), jnp.int32),
        pltpu.VMEM((_HALF,), jnp.float32),
        pltpu.VMEM((_B * _N,), jnp.float32),
        pltpu.VMEM((_N,), jnp.float32),
        pltpu.VMEM((_B * _N,), jnp.float32),
        pltpu.VMEM((_NT, _SLICE), jnp.float32),
        pltpu.VMEM((_SLICE,), jnp.float32),
        pltpu.VMEM_SHARED((_NT, _NPAD), jnp.float32),
        pltpu.VMEM_SHARED((_NPAD,), jnp.float32),
        pltpu.VMEM_SHARED((_NT, _B, _NPAD), jnp.float32),
    ],
)(_edge_body)


def _mv_body(x_ref, w_ref, z_ref):
    z_ref[...] = jnp.dot(x_ref[...], w_ref[...],
                         preferred_element_type=jnp.float32)


def _matvec(x2, wcat):
    return pl.pallas_call(
        _mv_body,
        grid=(20,),
        in_specs=[
            pl.BlockSpec((1000, _F), lambda i: (i, 0)),
            pl.BlockSpec((_F, 2), lambda i: (0, 0)),
        ],
        out_specs=pl.BlockSpec((1000, 2), lambda i: (i, 0)),
        out_shape=jax.ShapeDtypeStruct((_B * _N, 2), jnp.float32),
    )(x2, wcat)


def _asm_body(x_ref, z0_ref, t0_ref, t1_ref, b_ref, o_ref):
    sv = z0_ref[0, 0] + t0_ref[0, 0] + t1_ref[0, 0] + b_ref[0, 0]
    y = jax.nn.sigmoid(sv)
    o_ref[0, :, 0:_F] = x_ref[0]
    o_ref[0, :, _F:_F + 1] = y[:, None]


def _assemble(x, z0, t0, t1, bias):
    return pl.pallas_call(
        _asm_body,
        grid=(_B, _NB),
        in_specs=[
            pl.BlockSpec((1, _BN, _F), lambda b, j: (b, j, 0)),
            pl.BlockSpec((1, 1, _BN), lambda b, j: (b * _NB + j, 0, 0)),
            pl.BlockSpec((1, 1, _BN), lambda b, j: (b * _NB + j, 0, 0)),
            pl.BlockSpec((1, 1, _BN), lambda b, j: (b * _NB + j, 0, 0)),
            pl.BlockSpec((1, 1), lambda b, j: (0, 0)),
        ],
        out_specs=pl.BlockSpec((1, _BN, _F + 1), lambda b, j: (b, j, 0)),
        out_shape=jax.ShapeDtypeStruct((_B, _N, _F + 1), jnp.float32),
    )(x, z0, t0, t1, bias)


def kernel(x, edge_index, edge_w, W0, W1, bias):
    src = edge_index[0].astype(jnp.int32)
    dst = edge_index[1].astype(jnp.int32)
    x2 = x.reshape(_B * _N, _F)
    wcat = jnp.concatenate([W0, W1], axis=1)
    z = _matvec(x2, wcat)
    t = _edge_sc(src, dst, edge_w, z[:, 1])
    z0 = z[:, 0].reshape(_B * _NB, 1, _BN)
    t0 = t[0, :, :_N].reshape(_B * _NB, 1, _BN)
    t1 = t[1, :, :_N].reshape(_B * _NB, 1, _BN)
    b2 = bias.reshape(1, 1)
    return _assemble(x, z0, t0, t1, b2)


# speedup vs baseline: 352.5863x; 1.3544x over previous
import functools

import jax
import jax.numpy as jnp
from jax import lax
from jax.experimental import pallas as pl
from jax.experimental.pallas import tpu as pltpu
from jax.experimental.pallas import tpu_sc as plsc

_N = 10000
_E = 320000
_B = 2
_F = 128
_NT = 16
_NC = 2
_CHUNK = _E // _NT
_HALF = _CHUNK // _NC
_NPAD = 10240
_SLICE = _NPAD // _NT


def _rsqrt16(d):
    i = plsc.bitcast(d, jnp.int32)
    i = jnp.int32(0x5F3759DF) - (i >> 1)
    y = plsc.bitcast(i, jnp.float32)
    for _ in range(3):
        y = y * (1.5 - 0.5 * d * y * y)
    return jnp.where(d > 0.0, y, 0.0)


def _edge_body(ei_hbm, w_hbm, z1_hbm, t_hbm,
               src_v, dst_v, w_v, z1_v, dinv_v, t_v, red_v, slc_v, sem,
               deg_sh, dinv_sh, t_sh):
    c = lax.axis_index("c")
    s = lax.axis_index("s")

    z1_cp = pltpu.make_async_copy(z1_hbm, z1_v, sem)
    z1_cp.start()

    def _zero_deg(i, carry):
        t_v[pl.ds(i * 16, 16)] = jnp.zeros((16,), jnp.float32)
        return carry
    lax.fori_loop(0, _N // 16, _zero_deg, None)

    def _deg1(off):
        sv = src_v[pl.ds(off, 16)]
        dv = dst_v[pl.ds(off, 16)]
        wv = w_v[pl.ds(off, 16)]
        plsc.addupdate_scatter(t_v, [sv], jnp.where(sv != dv, wv, 0.0))

    def _deg(i, carry):
        for u in range(4):
            _deg1(i * 64 + u * 16)
        return carry

    for p in (1, 0):
        off = s * _CHUNK + (c ^ p) * _HALF
        pltpu.sync_copy(ei_hbm.at[0, pl.ds(off, _HALF)], src_v)
        pltpu.sync_copy(ei_hbm.at[1, pl.ds(off, _HALF)], dst_v)
        pltpu.sync_copy(w_hbm.at[pl.ds(off, _HALF)], w_v)
        lax.fori_loop(0, _HALF // 64, _deg, None)
        _deg1(_HALF - 16)

    pltpu.sync_copy(t_v.at[pl.ds(0, _N)], deg_sh.at[s, pl.ds(0, _N)])
    plsc.subcore_barrier()

    pltpu.sync_copy(deg_sh.at[:, pl.ds(s * _SLICE, _SLICE)], red_v)

    def _dinv(j, carry):
        acc = red_v[0, pl.ds(j * 16, 16)]
        for k in range(1, _NT):
            acc = acc + red_v[k, pl.ds(j * 16, 16)]
        slc_v[pl.ds(j * 16, 16)] = _rsqrt16(acc)
        return carry
    lax.fori_loop(0, _SLICE // 16, _dinv, None)

    pltpu.sync_copy(slc_v, dinv_sh.at[pl.ds(s * _SLICE, _SLICE)])
    plsc.subcore_barrier()
    pltpu.sync_copy(dinv_sh.at[pl.ds(0, _N)], dinv_v)

    def _zero_t(i, carry):
        t_v[pl.ds(i * 16, 16)] = jnp.zeros((16,), jnp.float32)
        return carry
    lax.fori_loop(0, (_B * _N) // 16, _zero_t, None)

    z1_cp.wait()

    def _msg1(off):
        sv = src_v[pl.ds(off, 16)]
        dv = dst_v[pl.ds(off, 16)]
        wv = w_v[pl.ds(off, 16)]
        dsrc = plsc.load_gather(dinv_v, [sv])
        ddst = plsc.load_gather(dinv_v, [dv])
        ce = jnp.where(sv != dv, -(dsrc * wv * ddst), 0.0)
        g0 = plsc.load_gather(z1_v, [sv])
        g1 = plsc.load_gather(z1_v, [sv + _N])
        plsc.addupdate_scatter(t_v, [dv], ce * g0)
        plsc.addupdate_scatter(t_v, [dv + _N], ce * g1)

    def _msg(i, carry):
        for u in range(2):
            _msg1(i * 32 + u * 16)
        return carry
    lax.fori_loop(0, _HALF // 32, _msg, None)
    _msg1(_HALF - 16)

    pltpu.sync_copy(t_v.at[pl.ds(0, _N)], t_sh.at[s, 0, pl.ds(0, _N)])
    pltpu.sync_copy(t_v.at[pl.ds(_N, _N)], t_sh.at[s, 1, pl.ds(0, _N)])
    plsc.subcore_barrier()

    for b in range(_B):
        pltpu.sync_copy(t_sh.at[:, b, pl.ds(s * _SLICE, _SLICE)], red_v)

        def _tred(j, carry):
            acc = red_v[0, pl.ds(j * 16, 16)]
            for k in range(1, _NT):
                acc = acc + red_v[k, pl.ds(j * 16, 16)]
            slc_v[pl.ds(j * 16, 16)] = acc
            return carry
        lax.fori_loop(0, _SLICE // 16, _tred, None)
        pltpu.sync_copy(slc_v, t_hbm.at[b, c, pl.ds(s * _SLICE, _SLICE)])


_edge_sc = functools.partial(
    pl.kernel,
    out_type=jax.ShapeDtypeStruct((_B, _NC, _NPAD), jnp.float32),
    mesh=plsc.VectorSubcoreMesh(core_axis_name="c", subcore_axis_name="s"),
    compiler_params=pltpu.CompilerParams(use_tc_tiling_on_sc=False,
                                         needs_layout_passes=False),
    scratch_types=[
        pltpu.VMEM((_HALF,), jnp.int32),
        pltpu.VMEM((_HALF,), jnp.int32),
        pltpu.VMEM((_HALF,), jnp.float32),
        pltpu.VMEM((_B * _N,), jnp.float32),
        pltpu.VMEM((_N,), jnp.float32),
        pltpu.VMEM((_B * _N,), jnp.float32),
        pltpu.VMEM((_NT, _SLICE), jnp.float32),
        pltpu.VMEM((_SLICE,), jnp.float32),
        pltpu.SemaphoreType.DMA,
        pltpu.VMEM_SHARED((_NT, _NPAD), jnp.float32),
        pltpu.VMEM_SHARED((_NPAD,), jnp.float32),
        pltpu.VMEM_SHARED((_NT, _B, _NPAD), jnp.float32),
    ],
)(_edge_body)


def _mv_body(x_ref, w_ref, z0_ref, z1_ref):
    z = jnp.dot(x_ref[...], w_ref[...], preferred_element_type=jnp.float32)
    z0_ref[0, 0] = z[:, 0]
    z1_ref[0, 0] = z[:, 1]


def _matvec(x2, wcat):
    return pl.pallas_call(
        _mv_body,
        grid=(10,),
        in_specs=[
            pl.BlockSpec((2000, _F), lambda i: (i, 0)),
            pl.BlockSpec((_F, 2), lambda i: (0, 0)),
        ],
        out_specs=[
            pl.BlockSpec((1, 1, 2000), lambda i: (i, 0, 0)),
            pl.BlockSpec((1, 1, 2000), lambda i: (i, 0, 0)),
        ],
        out_shape=[
            jax.ShapeDtypeStruct((10, 1, 2000), jnp.float32),
            jax.ShapeDtypeStruct((10, 1, 2000), jnp.float32),
        ],
    )(x2, wcat)


def _asm_body(x_ref, z0_ref, t_ref, b_ref, o_ref):
    sv = (z0_ref[0, 0] + t_ref[0, 0, :_N] + t_ref[0, 1, :_N] + b_ref[0])
    y = jax.nn.sigmoid(sv)
    o_ref[0, :, 0:_F] = x_ref[0]
    o_ref[0, :, _F:_F + 1] = y[:, None]


def _assemble(x, z0, t, bias):
    return pl.pallas_call(
        _asm_body,
        grid=(_B,),
        in_specs=[
            pl.BlockSpec((1, _N, _F), lambda b: (b, 0, 0)),
            pl.BlockSpec((1, 1, _N), lambda b: (b, 0, 0)),
            pl.BlockSpec((1, _NC, _NPAD), lambda b: (b, 0, 0)),
            pl.BlockSpec((1,), lambda b: (0,)),
        ],
        out_specs=pl.BlockSpec((1, _N, _F + 1), lambda b: (b, 0, 0)),
        out_shape=jax.ShapeDtypeStruct((_B, _N, _F + 1), jnp.float32),
    )(x, z0, t, bias)


def kernel(x, edge_index, edge_w, W0, W1, bias):
    ei = edge_index.astype(jnp.int32)
    x2 = x.reshape(_B * _N, _F)
    wcat = jnp.concatenate([W0, W1], axis=1)
    z0, z1 = _matvec(x2, wcat)
    t = _edge_sc(ei, edge_w, z1.reshape(_B * _N))
    return _assemble(x, z0.reshape(_B, 1, _N), t, bias)


# speedup vs baseline: 415.8519x; 1.1794x over previous
import functools

import jax
import jax.numpy as jnp
from jax import lax
from jax.experimental import pallas as pl
from jax.experimental.pallas import tpu as pltpu
from jax.experimental.pallas import tpu_sc as plsc

_N = 10000
_E = 320000
_B = 2
_F = 128
_NT = 16
_NC = 2
_CHUNK = _E // _NT
_HALF = _CHUNK // _NC
_PIECE = 2000
_NPP = _HALF // _PIECE
_NPAD = 10240
_SLICE = _NPAD // _NT


def _rsqrt16(d):
    i = plsc.bitcast(d, jnp.int32)
    i = jnp.int32(0x5F3759DF) - (i >> 1)
    y = plsc.bitcast(i, jnp.float32)
    for _ in range(3):
        y = y * (1.5 - 0.5 * d * y * y)
    return jnp.where(d > 0.0, y, 0.0)


def _edge_body(ei_hbm, w_hbm, z1_hbm, t_hbm,
               sA, dA, wA, sB, dB, wB, z1_v, dinv_v, t_v, red_v, slc_v,
               z1_sem, semA, semB, deg_sh, dinv_sh, t_sh):
    c = lax.axis_index("c")
    s = lax.axis_index("s")
    bufs = ((sA, dA, wA, semA), (sB, dB, wB, semB))

    def start_piece(q, off):
        sv, dv, wv, sem = bufs[q]
        cps = (pltpu.make_async_copy(ei_hbm.at[0, pl.ds(off, _PIECE)], sv, sem),
               pltpu.make_async_copy(ei_hbm.at[1, pl.ds(off, _PIECE)], dv, sem),
               pltpu.make_async_copy(w_hbm.at[pl.ds(off, _PIECE)], wv, sem))
        for cp in cps:
            cp.start()
        return cps

    def wait_piece(cps):
        for cp in cps:
            cp.wait()

    z1_cp = pltpu.make_async_copy(z1_hbm, z1_v, z1_sem)
    z1_cp.start()

    def deg_off(p):
        return s * _CHUNK + (c ^ (1 if p < _NPP else 0)) * _HALF \
            + (p % _NPP) * _PIECE

    pend = start_piece(0, deg_off(0))

    def _zero_deg(i, carry):
        for u in range(5):
            t_v[pl.ds(i * 80 + u * 16, 16)] = jnp.zeros((16,), jnp.float32)
        return carry
    lax.fori_loop(0, _N // 80, _zero_deg, None)

    def _make_scatter(q):
        sv, dv, wv, _ = bufs[q]

        def _deg(i, carry):
            for u in range(5):
                off = i * 80 + u * 16
                a = sv[pl.ds(off, 16)]
                b = dv[pl.ds(off, 16)]
                w16 = wv[pl.ds(off, 16)]
                plsc.addupdate_scatter(t_v, [a], jnp.where(a != b, w16, 0.0))
            return carry
        return _deg

    scatters = (_make_scatter(0), _make_scatter(1))

    for p in range(2 * _NPP):
        q = p & 1
        cur = pend
        if p + 1 < 2 * _NPP:
            pend = start_piece(q ^ 1, deg_off(p + 1))
        wait_piece(cur)
        lax.fori_loop(0, _PIECE // 80, scatters[q], None)

    pltpu.sync_copy(t_v.at[pl.ds(0, _N)], deg_sh.at[s, pl.ds(0, _N)])
    plsc.subcore_barrier()

    pltpu.sync_copy(deg_sh.at[:, pl.ds(s * _SLICE, _SLICE)], red_v)

    def _dinv(j, carry):
        acc = red_v[0, pl.ds(j * 16, 16)]
        for k in range(1, _NT):
            acc = acc + red_v[k, pl.ds(j * 16, 16)]
        slc_v[pl.ds(j * 16, 16)] = _rsqrt16(acc)
        return carry
    lax.fori_loop(0, _SLICE // 16, _dinv, None)

    pltpu.sync_copy(slc_v, dinv_sh.at[pl.ds(s * _SLICE, _SLICE)])
    plsc.subcore_barrier()
    pltpu.sync_copy(dinv_sh.at[pl.ds(0, _N)], dinv_v)

    def _zero_t(i, carry):
        for u in range(5):
            t_v[pl.ds(i * 80 + u * 16, 16)] = jnp.zeros((16,), jnp.float32)
        return carry
    lax.fori_loop(0, (_B * _N) // 80, _zero_t, None)

    z1_cp.wait()

    def _make_msg(q):
        sv, dv, wv, _ = bufs[q]

        def _msg(i, carry):
            for u in range(5):
                off = i * 80 + u * 16
                a = sv[pl.ds(off, 16)]
                b = dv[pl.ds(off, 16)]
                w16 = wv[pl.ds(off, 16)]
                dsrc = plsc.load_gather(dinv_v, [a])
                ddst = plsc.load_gather(dinv_v, [b])
                ce = jnp.where(a != b, -(dsrc * w16 * ddst), 0.0)
                g0 = plsc.load_gather(z1_v, [a])
                g1 = plsc.load_gather(z1_v, [a + _N])
                plsc.addupdate_scatter(t_v, [b], ce * g0)
                plsc.addupdate_scatter(t_v, [b + _N], ce * g1)
            return carry
        return _msg

    msgs = (_make_msg(0), _make_msg(1))
    base = s * _CHUNK + c * _HALF
    order = [_NPP - 1] + list(range(_NPP - 1))
    pend = None
    for j, piece in enumerate(order):
        q = (1 + j) & 1
        cur = pend
        if j + 1 < _NPP:
            pend = start_piece(q ^ 1, base + order[j + 1] * _PIECE)
        if cur is not None:
            wait_piece(cur)
        lax.fori_loop(0, _PIECE // 80, msgs[q], None)

    pltpu.sync_copy(t_v.at[pl.ds(0, _N)], t_sh.at[s, 0, pl.ds(0, _N)])
    pltpu.sync_copy(t_v.at[pl.ds(_N, _N)], t_sh.at[s, 1, pl.ds(0, _N)])
    plsc.subcore_barrier()

    for b in range(_B):
        pltpu.sync_copy(t_sh.at[:, b, pl.ds(s * _SLICE, _SLICE)], red_v)

        def _tred(j, carry):
            acc = red_v[0, pl.ds(j * 16, 16)]
            for k in range(1, _NT):
                acc = acc + red_v[k, pl.ds(j * 16, 16)]
            slc_v[pl.ds(j * 16, 16)] = acc
            return carry
        lax.fori_loop(0, _SLICE // 16, _tred, None)
        pltpu.sync_copy(slc_v, t_hbm.at[b, c, pl.ds(s * _SLICE, _SLICE)])


_edge_sc = functools.partial(
    pl.kernel,
    out_type=jax.ShapeDtypeStruct((_B, _NC, _NPAD), jnp.float32),
    mesh=plsc.VectorSubcoreMesh(core_axis_name="c", subcore_axis_name="s"),
    compiler_params=pltpu.CompilerParams(use_tc_tiling_on_sc=False,
                                         needs_layout_passes=False),
    scratch_types=[
        pltpu.VMEM((_PIECE,), jnp.int32),
        pltpu.VMEM((_PIECE,), jnp.int32),
        pltpu.VMEM((_PIECE,), jnp.float32),
        pltpu.VMEM((_PIECE,), jnp.int32),
        pltpu.VMEM((_PIECE,), jnp.int32),
        pltpu.VMEM((_PIECE,), jnp.float32),
        pltpu.VMEM((_B * _N,), jnp.float32),
        pltpu.VMEM((_N,), jnp.float32),
        pltpu.VMEM((_B * _N,), jnp.float32),
        pltpu.VMEM((_NT, _SLICE), jnp.float32),
        pltpu.VMEM((_SLICE,), jnp.float32),
        pltpu.SemaphoreType.DMA,
        pltpu.SemaphoreType.DMA,
        pltpu.SemaphoreType.DMA,
        pltpu.VMEM_SHARED((_NT, _NPAD), jnp.float32),
        pltpu.VMEM_SHARED((_NPAD,), jnp.float32),
        pltpu.VMEM_SHARED((_NT, _B, _NPAD), jnp.float32),
    ],
)(_edge_body)


def _mv_body(x_ref, w0_ref, w1_ref, z0_ref, z1_ref):
    xb = x_ref[...]
    z0 = jnp.dot(xb, w0_ref[...], preferred_element_type=jnp.float32)
    z1 = jnp.dot(xb, w1_ref[...], preferred_element_type=jnp.float32)
    z0_ref[0, 0] = z0[:, 0]
    z1_ref[0, 0] = z1[:, 0]


def _matvec(x2, w0, w1):
    return pl.pallas_call(
        _mv_body,
        grid=(5,),
        in_specs=[
            pl.BlockSpec((4000, _F), lambda i: (i, 0)),
            pl.BlockSpec((_F, 1), lambda i: (0, 0)),
            pl.BlockSpec((_F, 1), lambda i: (0, 0)),
        ],
        out_specs=[
            pl.BlockSpec((1, 1, 4000), lambda i: (i, 0, 0)),
            pl.BlockSpec((1, 1, 4000), lambda i: (i, 0, 0)),
        ],
        out_shape=[
            jax.ShapeDtypeStruct((5, 1, 4000), jnp.float32),
            jax.ShapeDtypeStruct((5, 1, 4000), jnp.float32),
        ],
    )(x2, w0, w1)


def _y_body(z0_ref, t_ref, b_ref, y_ref):
    sv = z0_ref[0, 0] + t_ref[0, 0, :_N] + t_ref[0, 1, :_N] + b_ref[0]
    y_ref[0, 0] = jax.nn.sigmoid(sv)


def _sigmoid_y(z0, t, bias):
    return pl.pallas_call(
        _y_body,
        grid=(_B,),
        in_specs=[
            pl.BlockSpec((1, 1, _N), lambda b: (b, 0, 0)),
            pl.BlockSpec((1, _NC, _NPAD), lambda b: (b, 0, 0)),
            pl.BlockSpec((1,), lambda b: (0,)),
        ],
        out_specs=pl.BlockSpec((1, 1, _N), lambda b: (b, 0, 0)),
        out_shape=jax.ShapeDtypeStruct((_B, 1, _N), jnp.float32),
    )(z0, t, bias)


def kernel(x, edge_index, edge_w, W0, W1, bias):
    ei = edge_index.astype(jnp.int32)
    x2 = x.reshape(_B * _N, _F)
    z0, z1 = _matvec(x2, W0, W1)
    t = _edge_sc(ei, edge_w, z1.reshape(_B * _N))
    y = _sigmoid_y(z0.reshape(_B, 1, _N), t, bias)
    return jnp.concatenate([x, y.reshape(_B, _N, 1)], axis=-1)
